# baseline (speedup 1.0000x reference)
import functools

import jax
import jax.numpy as jnp
from jax import lax
from jax.experimental import pallas as pl
from jax.experimental.pallas import tpu as pltpu
from jax.experimental.pallas import tpu_sc as plsc

B, L, D, H, C = 4096, 200, 64, 256, 32
V = 1_000_000
TB = 4096
NBLK = 123
HALF = NBLK * TB

_info = plsc.get_sparse_core_info()
NC, NS = _info.num_cores, _info.num_subcores
NW = NC * NS
SEQ_PER_W = B // NW
TIERS = ((0, 56), (56, 48), (104, 48), (152, 48))


def _sc_pool_sums(x_flat, seq_lengths, table):
    mesh = plsc.VectorSubcoreMesh(core_axis_name="c", subcore_axis_name="s")

    @functools.partial(
        pl.kernel,
        mesh=mesh,
        compiler_params=pltpu.CompilerParams(use_tc_tiling_on_sc=False,
                                             needs_layout_passes=False),
        out_type=jax.ShapeDtypeStruct((B * D,), jnp.float32),
        scratch_types=[
            pltpu.VMEM((SEQ_PER_W * L,), jnp.int32),
            pltpu.VMEM((SEQ_PER_W,), jnp.int32),
            pltpu.VMEM((L, D), jnp.bfloat16),
            pltpu.VMEM((L, D), jnp.bfloat16),
            pltpu.VMEM((SEQ_PER_W * D,), jnp.float32),
            pltpu.SemaphoreType.DMA,
            pltpu.SemaphoreType.DMA,
        ],
    )
    def k(x_hbm, len_hbm, table_hbm, out_hbm,
          x_v, len_v, rows_a, rows_b, sums_v, sem_a, sem_b):
        wid = lax.axis_index("s") * NC + lax.axis_index("c")
        base = wid * SEQ_PER_W
        pltpu.sync_copy(x_hbm.at[pl.ds(base * L, SEQ_PER_W * L)], x_v)
        pltpu.sync_copy(len_hbm.at[pl.ds(base, SEQ_PER_W)], len_v)

        def remap(g2, _):
            for u in range(4):
                off = g2 * 64 + u * 16
                v = x_v[pl.ds(off, 16)]
                x_v[pl.ds(off, 16)] = jnp.where(
                    v < HALF, v * 2, v * 2 - (2 * HALF - 1))
            return _

        lax.fori_loop(0, SEQ_PER_W * L // 64, remap, 0)

        def issue(b, n, rows_ref, sem):
            for st, sz in TIERS:
                def do(st=st, sz=sz):
                    pltpu.async_copy(
                        table_hbm.at[x_v.at[pl.ds(b * L + st, sz)]],
                        rows_ref.at[pl.ds(st, sz)], sem)
                if st == 0:
                    do()
                else:
                    pl.when(n > st)(do)

        def wait_for(n, rows_ref, sem):
            for st, sz in TIERS:
                def dw(st=st, sz=sz):
                    pltpu.make_async_copy(
                        table_hbm.at[pl.ds(0, sz)],
                        rows_ref.at[pl.ds(st, sz)], sem).wait()
                if st == 0:
                    dw()
                else:
                    pl.when(n > st)(dw)

        bufs = ((rows_a, sem_a), (rows_b, sem_b))
        lens_first = len_v[pl.ds(0, 16)]
        issue(0, lens_first[0], rows_a, sem_a)

        def per_group(g, _):
            lens16 = len_v[pl.ds(g * 16, 16)]
            lens_nx = len_v[pl.ds(jnp.minimum((g + 1) * 16, SEQ_PER_W - 16),
                                  16)]
            last_g = (SEQ_PER_W // 16) - 1
            n_wrap = jnp.where(g == last_g, lens_nx[15], lens_nx[0])
            for j in range(16):
                b = g * 16 + j
                rows_v, sem = bufs[j % 2]
                nrows_v, nsem = bufs[(j + 1) % 2]
                n_next = lens16[j + 1] if j < 15 else n_wrap
                issue(jnp.minimum(b + 1, SEQ_PER_W - 1), n_next,
                      nrows_v, nsem)
                wait_for(lens16[j], rows_v, sem)
                n = lens16[j]
                n4 = n // 4

                def row_chunks(ridx):
                    out = []
                    for hc in range(2):
                        h = rows_v[ridx, pl.ds(32 * hc, 32)]
                        a, bb = plsc.unpack(
                            h, format=plsc.PackFormat.INTERLEAVED,
                            preferred_element_type=jnp.float32)
                        out += [a, bb]
                    return out

                def acc4(r, carry):
                    s = list(carry)
                    r4 = r * 4
                    for k in range(4):
                        ch = row_chunks(r4 + k)
                        for c in range(4):
                            s[4 * k + c] = s[4 * k + c] + ch[c]
                    return tuple(s)

                z = jnp.zeros((16,), jnp.float32)
                s = lax.fori_loop(0, n4, acc4, (z,) * 16)
                s = list(s)
                for t in range(3):
                    idx = jnp.minimum(n4 * 4 + t, L - 1)
                    pred = (n4 * 4 + t) < n
                    ch = row_chunks(idx)
                    for c in range(4):
                        v = jnp.where(pred, ch[c],
                                      jnp.zeros((16,), jnp.float32))
                        s[4 * t + c] = s[4 * t + c] + v
                for c in range(4):
                    sums_v[pl.ds(b * D + 16 * c, 16)] = (
                        s[c] + s[4 + c] + s[8 + c] + s[12 + c])
            return _

        lax.fori_loop(0, SEQ_PER_W // 16, per_group, 0)
        lens_last = len_v[pl.ds(SEQ_PER_W - 16, 16)]
        wait_for(lens_last[15], rows_a, sem_a)
        pltpu.sync_copy(sums_v, out_hbm.at[pl.ds(base * D, SEQ_PER_W * D)])

    return k(x_flat, seq_lengths, table)


def _transpose_body(ina_ref, inb_ref, out_ref):
    r = jax.lax.broadcasted_iota(jnp.int32, (D, D), 0)
    c = jax.lax.broadcasted_iota(jnp.int32, (D, D), 1)
    eye = (r == c).astype(jnp.float32)
    dn = (((0,), (0,)), ((), ()))
    ya = jax.lax.dot_general(ina_ref[...], eye, dn,
                             preferred_element_type=jnp.float32)
    yb = jax.lax.dot_general(inb_ref[...], eye, dn,
                             preferred_element_type=jnp.float32)
    out_ref[...] = jnp.concatenate([ya, yb], axis=1).astype(jnp.bfloat16)


def _tc_transpose(tableT):
    return pl.pallas_call(
        _transpose_body,
        grid=(NBLK,),
        in_specs=[
            pl.BlockSpec((D, TB), lambda i: (0, i)),
            pl.BlockSpec((D, TB), lambda i: (0, jnp.minimum(i + NBLK,
                                                            V // TB))),
        ],
        out_specs=pl.BlockSpec((TB, 2 * D), lambda i: (i, 0)),
        out_shape=jax.ShapeDtypeStruct((HALF, 2 * D), jnp.bfloat16),
    )(tableT, tableT)


def _mlp_body(sums_ref, lens_ref, w1_ref, b1_ref, w2_ref, b2_ref, out_ref):
    pooled = sums_ref[...] / lens_ref[...]
    h = jnp.dot(pooled, w1_ref[...], preferred_element_type=jnp.float32)
    h = jnp.maximum(h + b1_ref[...], 0.0)
    o = jnp.dot(h, w2_ref[...], preferred_element_type=jnp.float32)
    out_ref[...] = o + b2_ref[...]


def _tc_mlp(sums, lens_f, W1, b1, W2, b2):
    blk = 512
    grid = B // blk
    return pl.pallas_call(
        _mlp_body,
        grid=(grid,),
        in_specs=[
            pl.BlockSpec((blk, D), lambda i: (i, 0)),
            pl.BlockSpec((blk, 1), lambda i: (i, 0)),
            pl.BlockSpec((D, H), lambda i: (0, 0)),
            pl.BlockSpec((1, H), lambda i: (0, 0)),
            pl.BlockSpec((H, C), lambda i: (0, 0)),
            pl.BlockSpec((1, C), lambda i: (0, 0)),
        ],
        out_specs=pl.BlockSpec((blk, C), lambda i: (i, 0)),
        out_shape=jax.ShapeDtypeStruct((B, C), jnp.float32),
    )(sums, lens_f, W1, b1, W2, b2)


_PERM = ([2 * i for i in range(16)] + [2 * i + 1 for i in range(16)]
         + [32 + 2 * i for i in range(16)] + [33 + 2 * i for i in range(16)])


def kernel(x, seq_lengths, table, W1, b1, W2, b2):
    x_flat = x.astype(jnp.int32).reshape(B * L)
    table_lin = _tc_transpose(table.T).reshape(2 * HALF, D)
    sums = _sc_pool_sums(x_flat, seq_lengths.astype(jnp.int32), table_lin)
    sums = sums.reshape(B, D)
    lens_f = seq_lengths.astype(jnp.float32).reshape(B, 1)
    W1p = W1[jnp.array(_PERM), :]
    return _tc_mlp(sums, lens_f, W1p, b1.reshape(1, H), W2, b2.reshape(1, C))


# speedup vs baseline: 1.8789x; 1.8789x over previous
import functools

import jax
import jax.numpy as jnp
from jax import lax
from jax.experimental import pallas as pl
from jax.experimental.pallas import tpu as pltpu
from jax.experimental.pallas import tpu_sc as plsc

B, L, D, H, C = 4096, 200, 64, 256, 32
V = 1_000_000
TB = 4096
NBLK = 123
HALF = NBLK * TB

_info = plsc.get_sparse_core_info()
NC, NS = _info.num_cores, _info.num_subcores
NW = NC * NS
SEQ_PER_W = B // NW
TIERS = ((0, 56), (56, 48), (104, 48), (152, 48))


def _sc_pool_sums(x_flat, seq_lengths, table):
    mesh = plsc.VectorSubcoreMesh(core_axis_name="c", subcore_axis_name="s")

    @functools.partial(
        pl.kernel,
        mesh=mesh,
        compiler_params=pltpu.CompilerParams(use_tc_tiling_on_sc=False),
        out_type=jax.ShapeDtypeStruct((B * D,), jnp.float32),
        scratch_types=[
            pltpu.VMEM((SEQ_PER_W * L,), jnp.int32),
            pltpu.VMEM((SEQ_PER_W,), jnp.int32),
            pltpu.VMEM((L, D), jnp.float32),
            pltpu.VMEM((L, D), jnp.float32),
            pltpu.VMEM((SEQ_PER_W * D,), jnp.float32),
            pltpu.SemaphoreType.DMA,
            pltpu.SemaphoreType.DMA,
        ],
    )
    def k(x_hbm, len_hbm, table_hbm, out_hbm,
          x_v, len_v, rows_a, rows_b, sums_v, sem_a, sem_b):
        wid = lax.axis_index("s") * NC + lax.axis_index("c")
        base = wid * SEQ_PER_W
        pltpu.sync_copy(x_hbm.at[pl.ds(base * L, SEQ_PER_W * L)], x_v)
        pltpu.sync_copy(len_hbm.at[pl.ds(base, SEQ_PER_W)], len_v)

        def remap(g2, _):
            for u in range(4):
                off = g2 * 64 + u * 16
                v = x_v[pl.ds(off, 16)]
                x_v[pl.ds(off, 16)] = jnp.where(
                    v < HALF, v * 2, v * 2 - (2 * HALF - 1))
            return _

        lax.fori_loop(0, SEQ_PER_W * L // 64, remap, 0)

        def issue(b, n, rows_ref, sem):
            for st, sz in TIERS:
                def do(st=st, sz=sz):
                    pltpu.async_copy(
                        table_hbm.at[x_v.at[pl.ds(b * L + st, sz)]],
                        rows_ref.at[pl.ds(st, sz)], sem)
                if st == 0:
                    do()
                else:
                    pl.when(n > st)(do)

        def wait_for(n, rows_ref, sem):
            for st, sz in TIERS:
                def dw(st=st, sz=sz):
                    pltpu.make_async_copy(
                        table_hbm.at[pl.ds(0, sz)],
                        rows_ref.at[pl.ds(st, sz)], sem).wait()
                if st == 0:
                    dw()
                else:
                    pl.when(n > st)(dw)

        bufs = ((rows_a, sem_a), (rows_b, sem_b))
        lens_first = len_v[pl.ds(0, 16)]
        issue(0, lens_first[0], rows_a, sem_a)

        def per_group(g, _):
            lens16 = len_v[pl.ds(g * 16, 16)]
            lens_nx = len_v[pl.ds(jnp.minimum((g + 1) * 16, SEQ_PER_W - 16),
                                  16)]
            last_g = (SEQ_PER_W // 16) - 1
            n_wrap = jnp.where(g == last_g, lens_nx[15], lens_nx[0])
            for j in range(16):
                b = g * 16 + j
                rows_v, sem = bufs[j % 2]
                nrows_v, nsem = bufs[(j + 1) % 2]
                n_next = lens16[j + 1] if j < 15 else n_wrap
                issue(jnp.minimum(b + 1, SEQ_PER_W - 1), n_next,
                      nrows_v, nsem)
                wait_for(lens16[j], rows_v, sem)
                n = lens16[j]
                n4 = n // 4

                def acc4(r, carry):
                    s = list(carry)
                    r4 = r * 4
                    for k in range(4):
                        for c in range(4):
                            s[4 * k + c] = (s[4 * k + c]
                                            + rows_v[r4 + k, pl.ds(16 * c, 16)])
                    return tuple(s)

                z = jnp.zeros((16,), jnp.float32)
                s = lax.fori_loop(0, n4, acc4, (z,) * 16)
                s = list(s)
                for t in range(3):
                    idx = jnp.minimum(n4 * 4 + t, L - 1)
                    pred = (n4 * 4 + t) < n
                    for c in range(4):
                        v = jnp.where(pred, rows_v[idx, pl.ds(16 * c, 16)],
                                      jnp.zeros((16,), jnp.float32))
                        s[4 * t + c] = s[4 * t + c] + v
                for c in range(4):
                    sums_v[pl.ds(b * D + 16 * c, 16)] = (
                        s[c] + s[4 + c] + s[8 + c] + s[12 + c])
            return _

        lax.fori_loop(0, SEQ_PER_W // 16, per_group, 0)
        lens_last = len_v[pl.ds(SEQ_PER_W - 16, 16)]
        wait_for(lens_last[15], rows_a, sem_a)
        pltpu.sync_copy(sums_v, out_hbm.at[pl.ds(base * D, SEQ_PER_W * D)])

    return k(x_flat, seq_lengths, table)


def _transpose_body(ina_ref, inb_ref, out_ref):
    r = jax.lax.broadcasted_iota(jnp.int32, (D, D), 0)
    c = jax.lax.broadcasted_iota(jnp.int32, (D, D), 1)
    eye = (r == c).astype(jnp.float32)
    dn = (((0,), (0,)), ((), ()))
    ya = jax.lax.dot_general(ina_ref[...], eye, dn,
                             preferred_element_type=jnp.float32)
    yb = jax.lax.dot_general(inb_ref[...], eye, dn,
                             preferred_element_type=jnp.float32)
    out_ref[...] = jnp.concatenate([ya, yb], axis=1)


def _tc_transpose(tableT):
    return pl.pallas_call(
        _transpose_body,
        grid=(NBLK,),
        in_specs=[
            pl.BlockSpec((D, TB), lambda i: (0, i)),
            pl.BlockSpec((D, TB), lambda i: (0, jnp.minimum(i + NBLK,
                                                            V // TB))),
        ],
        out_specs=pl.BlockSpec((TB, 2 * D), lambda i: (i, 0)),
        out_shape=jax.ShapeDtypeStruct((HALF, 2 * D), jnp.float32),
    )(tableT, tableT)


def _mlp_body(sums_ref, lens_ref, w1_ref, b1_ref, w2_ref, b2_ref, out_ref):
    pooled = sums_ref[...] / lens_ref[...]
    h = jnp.dot(pooled, w1_ref[...], preferred_element_type=jnp.float32)
    h = jnp.maximum(h + b1_ref[...], 0.0)
    o = jnp.dot(h, w2_ref[...], preferred_element_type=jnp.float32)
    out_ref[...] = o + b2_ref[...]


def _tc_mlp(sums, lens_f, W1, b1, W2, b2):
    blk = 512
    grid = B // blk
    return pl.pallas_call(
        _mlp_body,
        grid=(grid,),
        in_specs=[
            pl.BlockSpec((blk, D), lambda i: (i, 0)),
            pl.BlockSpec((blk, 1), lambda i: (i, 0)),
            pl.BlockSpec((D, H), lambda i: (0, 0)),
            pl.BlockSpec((1, H), lambda i: (0, 0)),
            pl.BlockSpec((H, C), lambda i: (0, 0)),
            pl.BlockSpec((1, C), lambda i: (0, 0)),
        ],
        out_specs=pl.BlockSpec((blk, C), lambda i: (i, 0)),
        out_shape=jax.ShapeDtypeStruct((B, C), jnp.float32),
    )(sums, lens_f, W1, b1, W2, b2)


def kernel(x, seq_lengths, table, W1, b1, W2, b2):
    x_flat = x.astype(jnp.int32).reshape(B * L)
    table_lin = _tc_transpose(table.T).reshape(2 * HALF, D)
    sums = _sc_pool_sums(x_flat, seq_lengths.astype(jnp.int32), table_lin)
    sums = sums.reshape(B, D)
    lens_f = seq_lengths.astype(jnp.float32).reshape(B, 1)
    return _tc_mlp(sums, lens_f, W1, b1.reshape(1, H), W2, b2.reshape(1, C))


# speedup vs baseline: 2.0510x; 1.0916x over previous
import functools

import jax
import jax.numpy as jnp
from jax import lax
from jax.experimental import pallas as pl
from jax.experimental.pallas import tpu as pltpu
from jax.experimental.pallas import tpu_sc as plsc

B, L, D, H, C = 4096, 200, 64, 256, 32
V = 1_000_000
TB = 8192
NBLK = 62
HALF = NBLK * TB

_info = plsc.get_sparse_core_info()
NC, NS = _info.num_cores, _info.num_subcores
NW = NC * NS
SEQ_PER_W = B // NW
TIERS = ((0, 56), (56, 48), (104, 48), (152, 48))


def _sc_pool_sums(x_flat, seq_lengths, table):
    mesh = plsc.VectorSubcoreMesh(core_axis_name="c", subcore_axis_name="s")

    @functools.partial(
        pl.kernel,
        mesh=mesh,
        compiler_params=pltpu.CompilerParams(use_tc_tiling_on_sc=False),
        out_type=jax.ShapeDtypeStruct((B * D,), jnp.float32),
        scratch_types=[
            pltpu.VMEM((SEQ_PER_W * L,), jnp.int32),
            pltpu.VMEM((SEQ_PER_W,), jnp.int32),
            pltpu.VMEM((L, D), jnp.float32),
            pltpu.VMEM((L, D), jnp.float32),
            pltpu.VMEM((SEQ_PER_W * D,), jnp.float32),
            pltpu.SemaphoreType.DMA,
            pltpu.SemaphoreType.DMA,
        ],
    )
    def k(x_hbm, len_hbm, table_hbm, out_hbm,
          x_v, len_v, rows_a, rows_b, sums_v, sem_a, sem_b):
        wid = lax.axis_index("s") * NC + lax.axis_index("c")
        base = wid * SEQ_PER_W
        pltpu.sync_copy(x_hbm.at[pl.ds(base * L, SEQ_PER_W * L)], x_v)
        pltpu.sync_copy(len_hbm.at[pl.ds(base, SEQ_PER_W)], len_v)

        def remap(g2, _):
            for u in range(4):
                off = g2 * 64 + u * 16
                v = x_v[pl.ds(off, 16)]
                x_v[pl.ds(off, 16)] = jnp.where(
                    v < HALF, v * 2, v * 2 - (2 * HALF - 1))
            return _

        lax.fori_loop(0, SEQ_PER_W * L // 64, remap, 0)

        def issue(b, n, rows_ref, sem):
            for st, sz in TIERS:
                def do(st=st, sz=sz):
                    pltpu.async_copy(
                        table_hbm.at[x_v.at[pl.ds(b * L + st, sz)]],
                        rows_ref.at[pl.ds(st, sz)], sem)
                if st == 0:
                    do()
                else:
                    pl.when(n > st)(do)

        def wait_for(n, rows_ref, sem):
            for st, sz in TIERS:
                def dw(st=st, sz=sz):
                    pltpu.make_async_copy(
                        table_hbm.at[pl.ds(0, sz)],
                        rows_ref.at[pl.ds(st, sz)], sem).wait()
                if st == 0:
                    dw()
                else:
                    pl.when(n > st)(dw)

        bufs = ((rows_a, sem_a), (rows_b, sem_b))
        lens_first = len_v[pl.ds(0, 16)]
        issue(0, lens_first[0], rows_a, sem_a)

        def per_group(g, _):
            lens16 = len_v[pl.ds(g * 16, 16)]
            lens_nx = len_v[pl.ds(jnp.minimum((g + 1) * 16, SEQ_PER_W - 16),
                                  16)]
            last_g = (SEQ_PER_W // 16) - 1
            n_wrap = jnp.where(g == last_g, lens_nx[15], lens_nx[0])
            for j in range(16):
                b = g * 16 + j
                rows_v, sem = bufs[j % 2]
                nrows_v, nsem = bufs[(j + 1) % 2]
                n_next = lens16[j + 1] if j < 15 else n_wrap
                issue(jnp.minimum(b + 1, SEQ_PER_W - 1), n_next,
                      nrows_v, nsem)
                wait_for(lens16[j], rows_v, sem)
                n = lens16[j]
                n4 = n // 4

                def acc4(r, carry):
                    s = list(carry)
                    r4 = r * 4
                    for k in range(4):
                        for c in range(4):
                            s[4 * k + c] = (s[4 * k + c]
                                            + rows_v[r4 + k, pl.ds(16 * c, 16)])
                    return tuple(s)

                z = jnp.zeros((16,), jnp.float32)
                s = lax.fori_loop(0, n4, acc4, (z,) * 16)
                s = list(s)
                for t in range(3):
                    idx = jnp.minimum(n4 * 4 + t, L - 1)
                    pred = (n4 * 4 + t) < n
                    for c in range(4):
                        v = jnp.where(pred, rows_v[idx, pl.ds(16 * c, 16)],
                                      jnp.zeros((16,), jnp.float32))
                        s[4 * t + c] = s[4 * t + c] + v
                for c in range(4):
                    sums_v[pl.ds(b * D + 16 * c, 16)] = (
                        s[c] + s[4 + c] + s[8 + c] + s[12 + c])
            return _

        lax.fori_loop(0, SEQ_PER_W // 16, per_group, 0)
        lens_last = len_v[pl.ds(SEQ_PER_W - 16, 16)]
        wait_for(lens_last[15], rows_a, sem_a)
        pltpu.sync_copy(sums_v, out_hbm.at[pl.ds(base * D, SEQ_PER_W * D)])

    return k(x_flat, seq_lengths, table)


def _transpose_body(ina_ref, inb_ref, out_ref):
    r = jax.lax.broadcasted_iota(jnp.int32, (D, D), 0)
    c = jax.lax.broadcasted_iota(jnp.int32, (D, D), 1)
    eye = (r == c).astype(jnp.float32)
    dn = (((0,), (0,)), ((), ()))
    ya = jax.lax.dot_general(ina_ref[...], eye, dn,
                             preferred_element_type=jnp.float32)
    yb = jax.lax.dot_general(inb_ref[...], eye, dn,
                             preferred_element_type=jnp.float32)
    out_ref[...] = jnp.concatenate([ya, yb], axis=1)


def _tc_transpose(tableT):
    return pl.pallas_call(
        _transpose_body,
        grid=(NBLK,),
        in_specs=[
            pl.BlockSpec((D, TB), lambda i: (0, i)),
            pl.BlockSpec((D, TB), lambda i: (0, jnp.minimum(i + NBLK,
                                                            V // TB))),
        ],
        out_specs=pl.BlockSpec((TB, 2 * D), lambda i: (i, 0)),
        out_shape=jax.ShapeDtypeStruct((HALF, 2 * D), jnp.float32),
    )(tableT, tableT)


def _mlp_body(sums_ref, lens_ref, w1_ref, b1_ref, w2_ref, b2_ref, out_ref):
    pooled = sums_ref[...] / lens_ref[...]
    h = jnp.dot(pooled, w1_ref[...], preferred_element_type=jnp.float32)
    h = jnp.maximum(h + b1_ref[...], 0.0)
    o = jnp.dot(h, w2_ref[...], preferred_element_type=jnp.float32)
    out_ref[...] = o + b2_ref[...]


def _tc_mlp(sums, lens_f, W1, b1, W2, b2):
    blk = 512
    grid = B // blk
    return pl.pallas_call(
        _mlp_body,
        grid=(grid,),
        in_specs=[
            pl.BlockSpec((blk, D), lambda i: (i, 0)),
            pl.BlockSpec((blk, 1), lambda i: (i, 0)),
            pl.BlockSpec((D, H), lambda i: (0, 0)),
            pl.BlockSpec((1, H), lambda i: (0, 0)),
            pl.BlockSpec((H, C), lambda i: (0, 0)),
            pl.BlockSpec((1, C), lambda i: (0, 0)),
        ],
        out_specs=pl.BlockSpec((blk, C), lambda i: (i, 0)),
        out_shape=jax.ShapeDtypeStruct((B, C), jnp.float32),
    )(sums, lens_f, W1, b1, W2, b2)


def kernel(x, seq_lengths, table, W1, b1, W2, b2):
    x_flat = x.astype(jnp.int32).reshape(B * L)
    table_lin = _tc_transpose(table.T).reshape(2 * HALF, D)
    sums = _sc_pool_sums(x_flat, seq_lengths.astype(jnp.int32), table_lin)
    sums = sums.reshape(B, D)
    lens_f = seq_lengths.astype(jnp.float32).reshape(B, 1)
    return _tc_mlp(sums, lens_f, W1, b1.reshape(1, H), W2, b2.reshape(1, C))


# speedup vs baseline: 2.1259x; 1.0365x over previous
import functools

import jax
import jax.numpy as jnp
from jax import lax
from jax.experimental import pallas as pl
from jax.experimental.pallas import tpu as pltpu
from jax.experimental.pallas import tpu_sc as plsc

B, L, D, H, C = 4096, 200, 64, 256, 32
V = 1_000_000
TB = 16384
NBLK = 31
HALF = NBLK * TB

_info = plsc.get_sparse_core_info()
NC, NS = _info.num_cores, _info.num_subcores
NW = NC * NS
SEQ_PER_W = B // NW
TIERS = ((0, 56), (56, 48), (104, 48), (152, 48))


def _sc_pool_sums(x_flat, seq_lengths, table):
    mesh = plsc.VectorSubcoreMesh(core_axis_name="c", subcore_axis_name="s")

    @functools.partial(
        pl.kernel,
        mesh=mesh,
        compiler_params=pltpu.CompilerParams(use_tc_tiling_on_sc=False),
        out_type=jax.ShapeDtypeStruct((B * D,), jnp.float32),
        scratch_types=[
            pltpu.VMEM((SEQ_PER_W * L,), jnp.int32),
            pltpu.VMEM((SEQ_PER_W,), jnp.int32),
            pltpu.VMEM((L, D), jnp.float32),
            pltpu.VMEM((L, D), jnp.float32),
            pltpu.VMEM((SEQ_PER_W * D,), jnp.float32),
            pltpu.SemaphoreType.DMA,
            pltpu.SemaphoreType.DMA,
        ],
    )
    def k(x_hbm, len_hbm, table_hbm, out_hbm,
          x_v, len_v, rows_a, rows_b, sums_v, sem_a, sem_b):
        wid = lax.axis_index("s") * NC + lax.axis_index("c")
        base = wid * SEQ_PER_W
        pltpu.sync_copy(x_hbm.at[pl.ds(base * L, SEQ_PER_W * L)], x_v)
        pltpu.sync_copy(len_hbm.at[pl.ds(base, SEQ_PER_W)], len_v)

        def remap(g2, _):
            for u in range(4):
                off = g2 * 64 + u * 16
                v = x_v[pl.ds(off, 16)]
                x_v[pl.ds(off, 16)] = jnp.where(
                    v < HALF, v * 2, v * 2 - (2 * HALF - 1))
            return _

        lax.fori_loop(0, SEQ_PER_W * L // 64, remap, 0)

        def issue(b, n, rows_ref, sem):
            for st, sz in TIERS:
                def do(st=st, sz=sz):
                    pltpu.async_copy(
                        table_hbm.at[x_v.at[pl.ds(b * L + st, sz)]],
                        rows_ref.at[pl.ds(st, sz)], sem)
                if st == 0:
                    do()
                else:
                    pl.when(n > st)(do)

        def wait_for(n, rows_ref, sem):
            for st, sz in TIERS:
                def dw(st=st, sz=sz):
                    pltpu.make_async_copy(
                        table_hbm.at[pl.ds(0, sz)],
                        rows_ref.at[pl.ds(st, sz)], sem).wait()
                if st == 0:
                    dw()
                else:
                    pl.when(n > st)(dw)

        bufs = ((rows_a, sem_a), (rows_b, sem_b))
        lens_first = len_v[pl.ds(0, 16)]
        issue(0, lens_first[0], rows_a, sem_a)

        def per_group(g, _):
            lens16 = len_v[pl.ds(g * 16, 16)]
            lens_nx = len_v[pl.ds(jnp.minimum((g + 1) * 16, SEQ_PER_W - 16),
                                  16)]
            last_g = (SEQ_PER_W // 16) - 1
            n_wrap = jnp.where(g == last_g, lens_nx[15], lens_nx[0])
            for j in range(16):
                b = g * 16 + j
                rows_v, sem = bufs[j % 2]
                nrows_v, nsem = bufs[(j + 1) % 2]
                n_next = lens16[j + 1] if j < 15 else n_wrap
                issue(jnp.minimum(b + 1, SEQ_PER_W - 1), n_next,
                      nrows_v, nsem)
                wait_for(lens16[j], rows_v, sem)
                n = lens16[j]
                n4 = n // 4

                def acc4(r, carry):
                    s = list(carry)
                    r4 = r * 4
                    for k in range(4):
                        for c in range(4):
                            s[4 * k + c] = (s[4 * k + c]
                                            + rows_v[r4 + k, pl.ds(16 * c, 16)])
                    return tuple(s)

                z = jnp.zeros((16,), jnp.float32)
                s = lax.fori_loop(0, n4, acc4, (z,) * 16)
                s = list(s)
                for t in range(3):
                    idx = jnp.minimum(n4 * 4 + t, L - 1)
                    pred = (n4 * 4 + t) < n
                    for c in range(4):
                        v = jnp.where(pred, rows_v[idx, pl.ds(16 * c, 16)],
                                      jnp.zeros((16,), jnp.float32))
                        s[4 * t + c] = s[4 * t + c] + v
                for c in range(4):
                    sums_v[pl.ds(b * D + 16 * c, 16)] = (
                        s[c] + s[4 + c] + s[8 + c] + s[12 + c])
            return _

        lax.fori_loop(0, SEQ_PER_W // 16, per_group, 0)
        lens_last = len_v[pl.ds(SEQ_PER_W - 16, 16)]
        wait_for(lens_last[15], rows_a, sem_a)
        pltpu.sync_copy(sums_v, out_hbm.at[pl.ds(base * D, SEQ_PER_W * D)])

    return k(x_flat, seq_lengths, table)


def _transpose_body(ina_ref, inb_ref, out_ref):
    r = jax.lax.broadcasted_iota(jnp.int32, (D, D), 0)
    c = jax.lax.broadcasted_iota(jnp.int32, (D, D), 1)
    eye = (r == c).astype(jnp.float32)
    dn = (((0,), (0,)), ((), ()))
    ya = jax.lax.dot_general(ina_ref[...], eye, dn,
                             preferred_element_type=jnp.float32)
    yb = jax.lax.dot_general(inb_ref[...], eye, dn,
                             preferred_element_type=jnp.float32)
    out_ref[...] = jnp.concatenate([ya, yb], axis=1)


def _tc_transpose(tableT):
    return pl.pallas_call(
        _transpose_body,
        grid=(NBLK,),
        in_specs=[
            pl.BlockSpec((D, TB), lambda i: (0, i)),
            pl.BlockSpec((D, TB), lambda i: (0, jnp.minimum(i + NBLK,
                                                            V // TB))),
        ],
        out_specs=pl.BlockSpec((TB, 2 * D), lambda i: (i, 0)),
        out_shape=jax.ShapeDtypeStruct((HALF, 2 * D), jnp.float32),
    )(tableT, tableT)


def _mlp_body(sums_ref, lens_ref, w1_ref, b1_ref, w2_ref, b2_ref, out_ref):
    pooled = sums_ref[...] / lens_ref[...]
    h = jnp.dot(pooled, w1_ref[...], preferred_element_type=jnp.float32)
    h = jnp.maximum(h + b1_ref[...], 0.0)
    o = jnp.dot(h, w2_ref[...], preferred_element_type=jnp.float32)
    out_ref[...] = o + b2_ref[...]


def _tc_mlp(sums, lens_f, W1, b1, W2, b2):
    blk = 512
    grid = B // blk
    return pl.pallas_call(
        _mlp_body,
        grid=(grid,),
        in_specs=[
            pl.BlockSpec((blk, D), lambda i: (i, 0)),
            pl.BlockSpec((blk, 1), lambda i: (i, 0)),
            pl.BlockSpec((D, H), lambda i: (0, 0)),
            pl.BlockSpec((1, H), lambda i: (0, 0)),
            pl.BlockSpec((H, C), lambda i: (0, 0)),
            pl.BlockSpec((1, C), lambda i: (0, 0)),
        ],
        out_specs=pl.BlockSpec((blk, C), lambda i: (i, 0)),
        out_shape=jax.ShapeDtypeStruct((B, C), jnp.float32),
    )(sums, lens_f, W1, b1, W2, b2)


def kernel(x, seq_lengths, table, W1, b1, W2, b2):
    x_flat = x.astype(jnp.int32).reshape(B * L)
    table_lin = _tc_transpose(table.T).reshape(2 * HALF, D)
    sums = _sc_pool_sums(x_flat, seq_lengths.astype(jnp.int32), table_lin)
    sums = sums.reshape(B, D)
    lens_f = seq_lengths.astype(jnp.float32).reshape(B, 1)
    return _tc_mlp(sums, lens_f, W1, b1.reshape(1, H), W2, b2.reshape(1, C))
